# baseline (speedup 1.0000x reference)
import functools

import jax
import jax.numpy as jnp
from jax import lax
from jax.experimental import pallas as pl
from jax.experimental.pallas import tpu as pltpu
from jax.experimental.pallas import tpu_sc as plsc

N_NODES = 10000
N_EDGES = 320000
D = 128
DH = D // 2

NC = 2
NS = 16
CHUNK = 128
NCHUNK = 160
EPT = NCHUNK * CHUNK
E_PAD = NS * EPT - N_EDGES
NSLOT = 4
GPC = CHUNK // 16
ROWS_MAIN = 624
ROWS_LAST = N_NODES - (NS - 1) * ROWS_MAIN
FBH = DH // 16


def _sc_spmm(x2, src, dst, w):
    mesh = plsc.VectorSubcoreMesh(core_axis_name="c", subcore_axis_name="s")

    @functools.partial(
        pl.kernel,
        mesh=mesh,
        compiler_params=pltpu.CompilerParams(use_tc_tiling_on_sc=False),
        out_type=jax.ShapeDtypeStruct((NC, N_NODES, DH), jnp.float32),
        scratch_types=[
            pltpu.VMEM((NCHUNK, CHUNK), jnp.int32),
            pltpu.VMEM((NCHUNK, CHUNK), jnp.int32),
            [pltpu.VMEM((CHUNK,), jnp.float32) for _ in range(NSLOT)],
            [pltpu.VMEM((CHUNK, DH), jnp.float32) for _ in range(NSLOT)],
            pltpu.VMEM_SHARED((N_NODES, DH), jnp.float32),
            [pltpu.SemaphoreType.DMA for _ in range(NSLOT)],
            [pltpu.SemaphoreType.DMA for _ in range(NSLOT)],
        ],
    )
    def spmm(x_hbm, src_hbm, dst_hbm, w_hbm, out_hbm,
             src_v, dst_v, wb, rowsb, acc_sh, gsem, ssem):
        c = lax.axis_index("c")
        s = lax.axis_index("s")
        row0 = pl.multiple_of(s * ROWS_MAIN, 8)

        zero16 = jnp.zeros((16,), jnp.float32)

        def zrow(r, carry):
            for f in range(FBH):
                rowsb[0][r, pl.ds(f * 16, 16)] = zero16
            return carry

        lax.fori_loop(0, CHUNK, zrow, 0)
        for k in range(ROWS_MAIN // CHUNK):
            pltpu.sync_copy(
                rowsb[0], acc_sh.at[pl.ds(row0 + k * CHUNK, CHUNK)])

        @pl.when(s < NS - 1)
        def _():
            pltpu.sync_copy(
                rowsb[0].at[pl.ds(0, ROWS_MAIN % CHUNK)],
                acc_sh.at[pl.ds(row0 + (ROWS_MAIN // CHUNK) * CHUNK,
                                ROWS_MAIN % CHUNK)])

        @pl.when(s == NS - 1)
        def _():
            for k in range(ROWS_MAIN // CHUNK, ROWS_LAST // CHUNK):
                pltpu.sync_copy(
                    rowsb[0], acc_sh.at[pl.ds(row0 + k * CHUNK, CHUNK)])

        plsc.subcore_barrier()

        pltpu.sync_copy(src_hbm.at[c, s], src_v)
        pltpu.sync_copy(dst_hbm.at[s], dst_v)

        def start_gather(ci, b):
            pltpu.async_copy(w_hbm.at[s, ci], wb[b], gsem[b])
            pltpu.async_copy(x_hbm.at[src_v.at[ci]], rowsb[b], gsem[b])

        def wait_gather(ci, b):
            pltpu.make_async_copy(w_hbm.at[s, ci], wb[b], gsem[b]).wait()
            pltpu.make_async_copy(
                x_hbm.at[src_v.at[ci]], rowsb[b], gsem[b]).wait()

        def start_scatter(ci, b):
            pltpu.async_copy(rowsb[b], acc_sh.at[dst_v.at[ci]], ssem[b],
                             add=True)

        def wait_scatter(ci, b):
            pltpu.make_async_copy(
                rowsb[b], acc_sh.at[dst_v.at[ci]], ssem[b]).wait()

        start_gather(0, 0)
        start_gather(1, 1)

        def quad_body(i, carry):
            for b in range(NSLOT):
                cur = i * NSLOT + b
                bg = (b + 2) % NSLOT

                @pl.when(cur + 2 < NCHUNK)
                def _():
                    @pl.when(cur >= 2)
                    def _():
                        wait_scatter(cur, bg)
                    start_gather(cur + 2, bg)

                wait_gather(cur, b)

                def mul_group(g, gcarry):
                    wvec = wb[b][pl.ds(g * 16, 16)]
                    for e in range(16):
                        wval = wvec[e]
                        r = g * 16 + e
                        for f in range(FBH):
                            blk = rowsb[b][r, pl.ds(f * 16, 16)]
                            rowsb[b][r, pl.ds(f * 16, 16)] = blk * wval
                    return gcarry

                lax.fori_loop(0, GPC, mul_group, 0)

                start_scatter(cur, b)
            return carry

        lax.fori_loop(0, NCHUNK // NSLOT, quad_body, 0)

        for b in range(NSLOT):
            wait_scatter(0, b)

        plsc.subcore_barrier()

        @pl.when(s < NS - 1)
        def _():
            pltpu.sync_copy(acc_sh.at[pl.ds(row0, ROWS_MAIN)],
                            out_hbm.at[c, pl.ds(row0, ROWS_MAIN)])

        @pl.when(s == NS - 1)
        def _():
            pltpu.sync_copy(acc_sh.at[pl.ds(row0, ROWS_LAST)],
                            out_hbm.at[c, pl.ds(row0, ROWS_LAST)])

    return spmm(x2, src, dst, w)


BR = 2000


def _combine_kernel(p_ref, w0_ref, w1_ref, b_ref, o_ref):
    o_ref[...] = (
        jnp.dot(p_ref[0], w0_ref[...], preferred_element_type=jnp.float32)
        + jnp.dot(p_ref[1], w1_ref[...], preferred_element_type=jnp.float32)
        + b_ref[...])


def _tc_combine(p, W0, W1, b2):
    return pl.pallas_call(
        _combine_kernel,
        grid=(N_NODES // BR,),
        in_specs=[
            pl.BlockSpec((NC, BR, DH), lambda i: (0, i, 0)),
            pl.BlockSpec((DH, D), lambda i: (0, 0)),
            pl.BlockSpec((DH, D), lambda i: (0, 0)),
            pl.BlockSpec((1, D), lambda i: (0, 0)),
        ],
        out_specs=pl.BlockSpec((BR, D), lambda i: (i, 0)),
        out_shape=jax.ShapeDtypeStruct((N_NODES, D), jnp.float32),
    )(p, W0, W1, b2)


def kernel(x, edge_index, edge_weight, W, b):
    x2 = jnp.concatenate([x[:, :DH], x[:, DH:]], axis=0)
    pad_idx = (jnp.arange(E_PAD, dtype=jnp.int32) * 8) % N_NODES
    src = jnp.concatenate(
        [edge_index[1].astype(jnp.int32), pad_idx]).reshape(NS, NCHUNK, CHUNK)
    src2 = jnp.stack([src, src + N_NODES])
    dst = jnp.concatenate(
        [edge_index[0].astype(jnp.int32), pad_idx]).reshape(NS, NCHUNK, CHUNK)
    w = jnp.concatenate(
        [edge_weight.astype(jnp.float32),
         jnp.zeros((E_PAD,), jnp.float32)]).reshape(NS, NCHUNK, CHUNK)
    p = _sc_spmm(x2, src2, dst, w)
    return _tc_combine(p, W[:DH], W[DH:], b.reshape(1, D))
